# Initial kernel scaffold; baseline (speedup 1.0000x reference)
import jax
import jax.numpy as jnp
from jax.experimental import pallas as pl


def kernel(x, edge_index, W1, b1, W2, b2):
    raise NotImplementedError("write your pallas kernel here")


# speedup vs baseline: 11.8011x; 11.8011x over previous
import functools

import jax
import jax.numpy as jnp
from jax import lax
from jax.experimental import pallas as pl
from jax.experimental.pallas import tpu as pltpu
from jax.experimental.pallas import tpu_sc as plsc

N = 10000
D = 128
E = 320000
NC = 2
NS = 16
NW = NC * NS
CH = 128
NCH = -(-E // (NW * CH))
E_PAD = NW * NCH * CH
ACC_ROWS = 10240
ROWS_PER_TILE = ACC_ROWS // NS
JUNK_ROW = 10200


def _deg_body(dst_hbm, zeros_hbm, out_hbm, idx_d, acc):
    c = lax.axis_index("c")
    s = lax.axis_index("s")
    wid = c * NS + s
    pltpu.sync_copy(dst_hbm.at[wid], idx_d)
    pltpu.sync_copy(zeros_hbm, acc)
    ones16 = jnp.ones((16,), jnp.float32)

    def body(j, carry):
        def inner(k, carry2):
            idx = idx_d[j, pl.ds(k * 16, 16)]
            plsc.addupdate_scatter(acc, [idx], ones16)
            return carry2
        return lax.fori_loop(0, CH // 16, inner, carry)

    lax.fori_loop(0, NCH, body, 0)
    pltpu.sync_copy(acc, out_hbm.at[wid])


def _agg_body(src_hbm, dst_hbm, p_hbm, zeros_hbm, out_hbm,
              idx_s, idx_d, rows, acc, sem):
    c = lax.axis_index("c")
    s = lax.axis_index("s")
    wid = c * NS + s
    pltpu.sync_copy(src_hbm.at[wid], idx_s)
    pltpu.sync_copy(dst_hbm.at[wid], idx_d)

    def zbody(k, carry):
        pltpu.sync_copy(
            zeros_hbm, acc.at[pl.ds(s * ROWS_PER_TILE + k * CH, CH)])
        return carry

    lax.fori_loop(0, ROWS_PER_TILE // CH, zbody, 0)
    plsc.subcore_barrier()

    def body(j, carry):
        pltpu.async_copy(p_hbm.at[idx_s.at[j]], rows, sem).wait()
        pltpu.sync_copy(rows, acc.at[idx_d.at[j]], add=True)
        return carry

    lax.fori_loop(0, NCH, body, 0)
    plsc.subcore_barrier()
    pltpu.sync_copy(
        acc.at[pl.ds(s * ROWS_PER_TILE, ROWS_PER_TILE)],
        out_hbm.at[c, pl.ds(s * ROWS_PER_TILE, ROWS_PER_TILE)],
    )


@functools.lru_cache(maxsize=None)
def _sc_kernels():
    mesh = plsc.VectorSubcoreMesh(core_axis_name="c", subcore_axis_name="s")
    deg_sc = pl.kernel(
        _deg_body,
        out_type=jax.ShapeDtypeStruct((NW, ACC_ROWS), jnp.float32),
        mesh=mesh,
        scratch_types=[
            pltpu.VMEM((NCH, CH), jnp.int32),
            pltpu.VMEM((ACC_ROWS,), jnp.float32),
        ],
        compiler_params=pltpu.CompilerParams(needs_layout_passes=False),
    )
    agg_sc = pl.kernel(
        _agg_body,
        out_type=jax.ShapeDtypeStruct((NC, ACC_ROWS, D), jnp.float32),
        mesh=mesh,
        scratch_types=[
            pltpu.VMEM((NCH, CH), jnp.int32),
            pltpu.VMEM((NCH, CH), jnp.int32),
            pltpu.VMEM((CH, D), jnp.float32),
            pltpu.VMEM_SHARED((ACC_ROWS, D), jnp.float32),
            pltpu.SemaphoreType.DMA,
        ],
    )
    return deg_sc, agg_sc


_BM = 1000
_NB = N // _BM


def _mm_body(x_ref, w_ref, o_ref):
    o_ref[...] = jnp.dot(x_ref[...], w_ref[...],
                         preferred_element_type=jnp.float32)


def _matmul_tc(x, w):
    return pl.pallas_call(
        _mm_body,
        grid=(_NB,),
        in_specs=[
            pl.BlockSpec((_BM, D), lambda i: (i, 0)),
            pl.BlockSpec((D, D), lambda i: (0, 0)),
        ],
        out_specs=pl.BlockSpec((_BM, D), lambda i: (i, 0)),
        out_shape=jax.ShapeDtypeStruct((N, D), jnp.float32),
    )(x, w)


def _prep_body(degp_ref, h_ref, dinv_ref, p_ref):
    deg = jnp.sum(degp_ref[...], axis=0) + 1.0
    dinv = lax.rsqrt(deg)
    dinv_ref[...] = dinv
    p_ref[...] = h_ref[...] * dinv


def _prep_tc(degp, h):
    return pl.pallas_call(
        _prep_body,
        grid=(_NB,),
        in_specs=[
            pl.BlockSpec((NW, _BM, 1), lambda i: (0, i, 0)),
            pl.BlockSpec((_BM, D), lambda i: (i, 0)),
        ],
        out_specs=[
            pl.BlockSpec((_BM, 1), lambda i: (i, 0)),
            pl.BlockSpec((_BM, D), lambda i: (i, 0)),
        ],
        out_shape=[
            jax.ShapeDtypeStruct((N, 1), jnp.float32),
            jax.ShapeDtypeStruct((N, D), jnp.float32),
        ],
    )(degp, h)


def _mid_body(a_ref, p_ref, dinv_ref, b_ref, w_ref, o_ref):
    t = dinv_ref[...] * (a_ref[0] + a_ref[1] + p_ref[...]) + b_ref[...]
    t = jnp.maximum(t, 0.0)
    o_ref[...] = jnp.dot(t, w_ref[...],
                         preferred_element_type=jnp.float32) * dinv_ref[...]


def _mid_tc(aggp, p, dinv, b, w):
    return pl.pallas_call(
        _mid_body,
        grid=(_NB,),
        in_specs=[
            pl.BlockSpec((NC, _BM, D), lambda i: (0, i, 0)),
            pl.BlockSpec((_BM, D), lambda i: (i, 0)),
            pl.BlockSpec((_BM, 1), lambda i: (i, 0)),
            pl.BlockSpec((1, D), lambda i: (0, 0)),
            pl.BlockSpec((D, D), lambda i: (0, 0)),
        ],
        out_specs=pl.BlockSpec((_BM, D), lambda i: (i, 0)),
        out_shape=jax.ShapeDtypeStruct((N, D), jnp.float32),
    )(aggp, p, dinv, b, w)


def _fin_body(a_ref, p_ref, dinv_ref, b_ref, o_ref):
    t = dinv_ref[...] * (a_ref[0] + a_ref[1] + p_ref[...]) + b_ref[...]
    o_ref[...] = jax.nn.sigmoid(t)


def _fin_tc(aggp, p, dinv, b):
    return pl.pallas_call(
        _fin_body,
        grid=(_NB,),
        in_specs=[
            pl.BlockSpec((NC, _BM, D), lambda i: (0, i, 0)),
            pl.BlockSpec((_BM, D), lambda i: (i, 0)),
            pl.BlockSpec((_BM, 1), lambda i: (i, 0)),
            pl.BlockSpec((1, D), lambda i: (0, 0)),
        ],
        out_specs=pl.BlockSpec((_BM, D), lambda i: (i, 0)),
        out_shape=jax.ShapeDtypeStruct((N, D), jnp.float32),
    )(aggp, p, dinv, b)


def kernel(x, edge_index, W1, b1, W2, b2):
    ei = edge_index.astype(jnp.int32)
    pad = E_PAD - E
    src = jnp.concatenate([ei[0], jnp.zeros((pad,), jnp.int32)])
    dst = jnp.concatenate([ei[1], jnp.full((pad,), JUNK_ROW, jnp.int32)])
    src_r = src.reshape(NW, NCH, CH)
    dst_r = dst.reshape(NW, NCH, CH)

    zeros_deg = jnp.zeros((ACC_ROWS,), jnp.float32)
    zeros_rows = jnp.zeros((CH, D), jnp.float32)

    deg_sc, agg_sc = _sc_kernels()
    degp = deg_sc(dst_r, zeros_deg)
    h1 = _matmul_tc(x, W1)
    dinv, p1 = _prep_tc(degp.reshape(NW, ACC_ROWS, 1), h1)

    agg1 = agg_sc(src_r, dst_r, p1, zeros_rows)
    p2 = _mid_tc(agg1, p1, dinv, b1[None, :], W2)

    agg2 = agg_sc(src_r, dst_r, p2, zeros_rows)
    return _fin_tc(agg2, p2, dinv, b2[None, :])
